# baseline (speedup 1.0000x reference)
import functools

import jax
import jax.numpy as jnp
from jax import lax
from jax.experimental import pallas as pl
from jax.experimental.pallas import tpu as pltpu
from jax.experimental.pallas import tpu_sc as plsc

MAX_REL = 128
VOCAB = 2 * MAX_REL + 1
HEAD_DIM = 64
SEQ = 2048
NUM_CORES = 2
NUM_SUBCORES = 16
NUM_WORKERS = NUM_CORES * NUM_SUBCORES
I_PER_W = SEQ // NUM_WORKERS
J_HALF = SEQ // 2
STRIP = J_HALF + I_PER_W - 1
STRIP_PAD = 1088
LANES = 16
CHUNKS = HEAD_DIM // LANES


def _rpe_body(table_hbm, out_hbm, table_v, strip_v, sem):
    wid = lax.axis_index("s") * NUM_CORES + lax.axis_index("c")
    iw = wid * I_PER_W

    pltpu.sync_copy(table_hbm, table_v)

    for half in range(2):
        j0 = half * J_HALF
        base = (I_PER_W - 1) + iw - j0

        def build_row(r, _):
            idx = jnp.clip(base - r, -MAX_REL, MAX_REL) + MAX_REL
            for ch in range(CHUNKS):
                strip_v[pl.ds(r * HEAD_DIM + ch * LANES, LANES)] = (
                    table_v[pl.ds(idx * HEAD_DIM + ch * LANES, LANES)])
            return 0

        lax.fori_loop(0, STRIP_PAD, build_row, 0)

        def copy_row(rr, _):
            off = (I_PER_W - 1) - rr
            dst = (iw + rr) * (SEQ * HEAD_DIM) + j0 * HEAD_DIM
            pltpu.async_copy(
                strip_v.at[pl.ds(off * HEAD_DIM, J_HALF * HEAD_DIM)],
                out_hbm.at[pl.ds(dst, J_HALF * HEAD_DIM)],
                sem)
            return 0

        lax.fori_loop(0, I_PER_W, copy_row, 0)

        def drain_row(rr, _):
            pltpu.make_async_copy(
                strip_v.at[pl.ds(0, J_HALF * HEAD_DIM)],
                out_hbm.at[pl.ds(iw * SEQ * HEAD_DIM, J_HALF * HEAD_DIM)],
                sem).wait()
            return 0

        lax.fori_loop(0, I_PER_W, drain_row, 0)


_rpe = functools.partial(
    pl.kernel,
    out_type=jax.ShapeDtypeStruct((SEQ * SEQ * HEAD_DIM,), jnp.float32),
    mesh=plsc.VectorSubcoreMesh(core_axis_name="c", subcore_axis_name="s"),
    scratch_types=[
        pltpu.VMEM((VOCAB * HEAD_DIM,), jnp.float32),
        pltpu.VMEM((STRIP_PAD * HEAD_DIM,), jnp.float32),
        pltpu.SemaphoreType.DMA,
    ],
)(_rpe_body)


def kernel(table, seq_len):
    del seq_len
    flat = _rpe(table.reshape(VOCAB * HEAD_DIM))
    return flat.reshape(SEQ, SEQ, HEAD_DIM)


# speedup vs baseline: 6.8547x; 6.8547x over previous
import functools

import jax
import jax.numpy as jnp
from jax import lax
from jax.experimental import pallas as pl
from jax.experimental.pallas import tpu as pltpu
from jax.experimental.pallas import tpu_sc as plsc

MAX_REL = 128
VOCAB = 2 * MAX_REL + 1
HEAD_DIM = 64
SEQ = 2048
NUM_CORES = 2
NUM_SUBCORES = 16
NUM_WORKERS = NUM_CORES * NUM_SUBCORES
LANES = 16

HT = HEAD_DIM // 8
JT = SEQ // 128
BLOCK = 8 * 128
NBLK = 2 * JT - 1
OUT_BLOCK = JT * BLOCK
ROW_WORDS = SEQ * HEAD_DIM
I0_PER_W = 128 // (NUM_WORKERS // HT)


def _rpe_body(table_hbm, out_hbm, table_v, g_v, sem):
    wid = lax.axis_index("s") * NUM_CORES + lax.axis_index("c")
    ht = wid % HT
    i0_base = (wid // HT) * I0_PER_W
    hbase = ht * 8

    pltpu.sync_copy(table_hbm, table_v)

    lane = lax.iota(jnp.int32, LANES)

    v256 = [plsc.load_gather(
        table_v, [jnp.full((LANES,), 256 * HEAD_DIM + hbase, jnp.int32) + h])
        for h in range(8)]
    v0 = [plsc.load_gather(
        table_v, [jnp.full((LANES,), hbase, jnp.int32) + h])
        for h in range(8)]

    def fill256(b, _):
        for h in range(8):
            for j16 in range(8):
                g_v[pl.ds(b * BLOCK + h * 128 + j16 * LANES, LANES)] = v256[h]
        return 0

    lax.fori_loop(0, JT - 2, fill256, 0)

    def fill0(b, _):
        for h in range(8):
            for j16 in range(8):
                g_v[pl.ds(b * BLOCK + h * 128 + j16 * LANES, LANES)] = v0[h]
        return 0

    lax.fori_loop(JT + 1, NBLK, fill0, 0)

    def per_i0(t, _):
        i0 = i0_base + t
        for m in (-1, 0, 1):
            b = (JT - 1) + m
            dbase = i0 - 128 * m
            for j16 in range(8):
                d = dbase - j16 * LANES - lane
                idx = jnp.clip(d, -MAX_REL, MAX_REL) + MAX_REL
                rowoff = idx * HEAD_DIM + hbase
                for h in range(8):
                    g_v[pl.ds(b * BLOCK + h * 128 + j16 * LANES, LANES)] = (
                        plsc.load_gather(table_v, [rowoff + h]))

        def fire(k, _):
            pltpu.async_copy(
                g_v.at[pl.ds((JT - 1 - k) * BLOCK, OUT_BLOCK)],
                out_hbm.at[pl.ds((i0 + 128 * k) * ROW_WORDS + ht * OUT_BLOCK,
                                 OUT_BLOCK)],
                sem)
            return 0

        lax.fori_loop(0, JT, fire, 0)

        def drain(k, _):
            pltpu.make_async_copy(
                g_v.at[pl.ds(0, OUT_BLOCK)],
                out_hbm.at[pl.ds(0, OUT_BLOCK)],
                sem).wait()
            return 0

        lax.fori_loop(0, JT, drain, 0)
        return 0

    lax.fori_loop(0, I0_PER_W, per_i0, 0)


_rpe = functools.partial(
    pl.kernel,
    out_type=jax.ShapeDtypeStruct((SEQ * SEQ * HEAD_DIM,), jnp.float32),
    mesh=plsc.VectorSubcoreMesh(core_axis_name="c", subcore_axis_name="s"),
    compiler_params=pltpu.CompilerParams(needs_layout_passes=False),
    scratch_types=[
        pltpu.VMEM((VOCAB * HEAD_DIM,), jnp.float32),
        pltpu.VMEM((NBLK * BLOCK,), jnp.float32),
        pltpu.SemaphoreType.DMA,
    ],
)(_rpe_body)


def kernel(table, seq_len):
    del seq_len
    flat = _rpe(table.reshape(VOCAB * HEAD_DIM))
    a = flat.reshape(SEQ, HT, JT, 8, 128)
    return a.transpose(0, 2, 4, 1, 3).reshape(SEQ, SEQ, HEAD_DIM)


# speedup vs baseline: 7.5752x; 1.1051x over previous
import functools

import jax
import jax.numpy as jnp
from jax import lax
from jax.experimental import pallas as pl
from jax.experimental.pallas import tpu as pltpu
from jax.experimental.pallas import tpu_sc as plsc

MAX_REL = 128
VOCAB = 2 * MAX_REL + 1
HEAD_DIM = 64
SEQ = 2048
NUM_CORES = 2
NUM_SUBCORES = 16
NUM_WORKERS = NUM_CORES * NUM_SUBCORES
LANES = 16

HT = HEAD_DIM // 8
JT = SEQ // 128
BLOCK = 8 * 128
NBLK = 2 * JT - 1
OUT_BLOCK = JT * BLOCK
ROW_WORDS = SEQ * HEAD_DIM
I0_PER_W = 128 // (NUM_WORKERS // HT)


def _rpe_body(table_hbm, out_hbm, table_v, g0_v, g1_v, sem0, sem1):
    wid = lax.axis_index("s") * NUM_CORES + lax.axis_index("c")
    ht = wid % HT
    i0_base = (wid // HT) * I0_PER_W
    hbase = ht * 8

    pltpu.sync_copy(table_hbm, table_v)

    lane = lax.iota(jnp.int32, LANES)
    bufs = ((g0_v, sem0), (g1_v, sem1))

    v256 = [plsc.load_gather(
        table_v, [jnp.full((LANES,), 256 * HEAD_DIM + hbase, jnp.int32) + h])
        for h in range(8)]
    v0 = [plsc.load_gather(
        table_v, [jnp.full((LANES,), hbase, jnp.int32) + h])
        for h in range(8)]

    for g_v, _ in bufs:
        def fill256(b, _, g_v=g_v):
            for h in range(8):
                for j16 in range(8):
                    g_v[pl.ds(b * BLOCK + h * 128 + j16 * LANES, LANES)] = (
                        v256[h])
            return 0

        lax.fori_loop(0, JT - 2, fill256, 0)

        def fill0(b, _, g_v=g_v):
            for h in range(8):
                for j16 in range(8):
                    g_v[pl.ds(b * BLOCK + h * 128 + j16 * LANES, LANES)] = (
                        v0[h])
            return 0

        lax.fori_loop(JT + 1, NBLK, fill0, 0)

    def drain16(g_v, sem):
        def drain(k, _):
            pltpu.make_async_copy(
                g_v.at[pl.ds(0, OUT_BLOCK)],
                out_hbm.at[pl.ds(0, OUT_BLOCK)],
                sem).wait()
            return 0

        lax.fori_loop(0, JT, drain, 0)

    def per_pair(tt, _):
        for p, (g_v, sem) in enumerate(bufs):
            i0 = i0_base + 2 * tt + p

            @pl.when(tt > 0)
            def _(g_v=g_v, sem=sem):
                drain16(g_v, sem)

            for m in (-1, 0, 1):
                b = (JT - 1) + m
                dbase = i0 - 128 * m
                for j16 in range(8):
                    d = dbase - j16 * LANES - lane
                    idx = jnp.clip(d, -MAX_REL, MAX_REL) + MAX_REL
                    rowoff = idx * HEAD_DIM + hbase
                    for h in range(8):
                        g_v[pl.ds(b * BLOCK + h * 128 + j16 * LANES,
                                  LANES)] = (
                            plsc.load_gather(table_v, [rowoff + h]))

            def fire(k, _, g_v=g_v, sem=sem, i0=i0):
                pltpu.async_copy(
                    g_v.at[pl.ds((JT - 1 - k) * BLOCK, OUT_BLOCK)],
                    out_hbm.at[pl.ds(
                        (i0 + 128 * k) * ROW_WORDS + ht * OUT_BLOCK,
                        OUT_BLOCK)],
                    sem)
                return 0

            lax.fori_loop(0, JT, fire, 0)
        return 0

    lax.fori_loop(0, I0_PER_W // 2, per_pair, 0)

    for g_v, sem in bufs:
        drain16(g_v, sem)


_rpe = functools.partial(
    pl.kernel,
    out_type=jax.ShapeDtypeStruct((SEQ * SEQ * HEAD_DIM,), jnp.float32),
    mesh=plsc.VectorSubcoreMesh(core_axis_name="c", subcore_axis_name="s"),
    compiler_params=pltpu.CompilerParams(needs_layout_passes=False),
    scratch_types=[
        pltpu.VMEM((VOCAB * HEAD_DIM,), jnp.float32),
        pltpu.VMEM((NBLK * BLOCK,), jnp.float32),
        pltpu.VMEM((NBLK * BLOCK,), jnp.float32),
        pltpu.SemaphoreType.DMA,
        pltpu.SemaphoreType.DMA,
    ],
)(_rpe_body)


def kernel(table, seq_len):
    del seq_len
    flat = _rpe(table.reshape(VOCAB * HEAD_DIM))
    a = flat.reshape(SEQ, HT, JT, 8, 128)
    return a.transpose(0, 2, 4, 1, 3).reshape(SEQ, SEQ, HEAD_DIM)
